# Initial kernel scaffold; baseline (speedup 1.0000x reference)
import jax
import jax.numpy as jnp
from jax.experimental import pallas as pl


def kernel(emb, W1, b1, W2, b2, tn_g, tn_b, Wt1, bt1, Wt2, bt2, cn_g, cn_b, Wc1, bc1, Wc2, bc2, node_t, edge_index_t):
    raise NotImplementedError("write your pallas kernel here")


# speedup vs baseline: 14.3629x; 14.3629x over previous
import functools

import jax
import jax.numpy as jnp
import numpy as np
from jax import lax
from jax.experimental import pallas as pl
from jax.experimental.pallas import tpu as pltpu
from jax.experimental.pallas import tpu_sc as plsc

N = 10000
E = 320000
T = 4
M = 8
D = 128

NPAD = 10240
NC, NS = 2, 16
NW = NC * NS
EPW = E // NW
CH = 80
NCHUNK = EPW // CH
ZROWS = NPAD // NS

NP = NPAD
SEG = 6
DM_TOT = 3 * E + 3 * 5000
DM_PAD = (-DM_TOT) % (NW * CH)
DM_PW = (DM_TOT + DM_PAD) // NW
DM_NCHUNK = DM_PW // CH
DM_ROWS = SEG * NP // NS
JUNK = 5 * NP + N

BN = 1024
NB = NPAD // BN

def _sc_mesh():
    return plsc.VectorSubcoreMesh(core_axis_name="c", subcore_axis_name="s",
                                  num_cores=NC, num_subcores=NS)


@functools.cache
def _build_sc_degmask():
    @functools.partial(
        pl.kernel,
        out_type=jax.ShapeDtypeStruct((NC * SEG * NP,), jnp.float32),
        mesh=_sc_mesh(),
        scratch_types=[
            pltpu.VMEM((DM_NCHUNK, CH), jnp.int32),
            pltpu.VMEM((CH,), jnp.float32),
            pltpu.VMEM((DM_ROWS,), jnp.float32),
            pltpu.VMEM_SHARED((SEG * NP,), jnp.float32),
            pltpu.SemaphoreType.DMA,
        ],
    )
    def _sc_degmask(idx_hbm, ones_hbm, zero_hbm, out_hbm,
                    idx_v, ones_v, stage_v, tab_sh, sem):
        c = lax.axis_index("c")
        s = lax.axis_index("s")
        wid = c * NS + s
        pltpu.sync_copy(zero_hbm, stage_v)
        pltpu.sync_copy(stage_v, tab_sh.at[pl.ds(s * DM_ROWS, DM_ROWS)])
        pltpu.sync_copy(ones_hbm, ones_v)
        pltpu.sync_copy(idx_hbm.at[wid], idx_v)
        plsc.subcore_barrier()

        def body(j, carry):
            pltpu.sync_copy(ones_v, tab_sh.at[idx_v.at[j]], add=True)
            return carry

        lax.fori_loop(0, DM_NCHUNK, body, 0)
        plsc.subcore_barrier()
        pltpu.sync_copy(tab_sh.at[pl.ds(s * DM_ROWS, DM_ROWS)], stage_v)
        pltpu.sync_copy(stage_v,
                        out_hbm.at[pl.ds(c * (SEG * NP) + s * DM_ROWS, DM_ROWS)])

    return _sc_degmask


@functools.cache
def _build_sc_aggregate():
    @functools.partial(
        pl.kernel,
        out_type=jax.ShapeDtypeStruct((NC * NPAD, D), jnp.float32),
        mesh=_sc_mesh(),
        scratch_types=[
            pltpu.VMEM((NCHUNK, CH), jnp.int32),
            pltpu.VMEM((NCHUNK, CH), jnp.int32),
            pltpu.VMEM((CH, D), jnp.float32),
            pltpu.VMEM_SHARED((NPAD, D), jnp.float32),
            pltpu.SemaphoreType.DMA,
        ],
    )
    def _sc_aggregate(table_hbm, src_hbm, dst_hbm, zero_hbm, out_hbm,
                      src_v, dst_v, rows_v, acc_sh, sem):
        c = lax.axis_index("c")
        s = lax.axis_index("s")
        wid = c * NS + s
        pltpu.sync_copy(zero_hbm, rows_v)
        for r in range(ZROWS // CH):
            pltpu.sync_copy(rows_v, acc_sh.at[pl.ds(s * ZROWS + r * CH, CH)])
        pltpu.sync_copy(src_hbm.at[wid], src_v)
        pltpu.sync_copy(dst_hbm.at[wid], dst_v)
        plsc.subcore_barrier()

        def body(j, carry):
            pltpu.async_copy(table_hbm.at[src_v.at[j]], rows_v, sem).wait()
            pltpu.sync_copy(rows_v, acc_sh.at[dst_v.at[j]], add=True)
            return carry

        lax.fori_loop(0, NCHUNK, body, 0)
        plsc.subcore_barrier()
        pltpu.sync_copy(acc_sh.at[pl.ds(s * ZROWS, ZROWS)],
                        out_hbm.at[pl.ds(c * NPAD + s * ZROWS, ZROWS)])

    return _sc_aggregate


_SQRT1_2 = np.float32(1.0 / np.sqrt(2.0))


def _gelu(x):
    return x * 0.5 * (1.0 + lax.erf(x * _SQRT1_2))


def _ln(x, g, b):
    mu = jnp.mean(x, axis=-1, keepdims=True)
    var = jnp.mean((x - mu) ** 2, axis=-1, keepdims=True)
    return (x - mu) * lax.rsqrt(var + np.float32(1e-5)) * g + b


def _dinv_from(dm0, dm1):
    cnt = dm0 + dm1
    return lax.rsqrt(cnt + 1.0).reshape(-1, 1)


def _mm_kernel(x_ref, w_ref, o_ref):
    o_ref[...] = jnp.dot(x_ref[...], w_ref[...], preferred_element_type=jnp.float32)


def _matmul(x, w):
    n = x.shape[0]
    return pl.pallas_call(
        _mm_kernel,
        grid=(n // BN,),
        in_specs=[pl.BlockSpec((BN, D), lambda i: (i, 0)),
                  pl.BlockSpec((D, D), lambda i: (0, 0))],
        out_specs=pl.BlockSpec((BN, D), lambda i: (i, 0)),
        out_shape=jax.ShapeDtypeStruct((n, D), jnp.float32),
    )(x, w)


def _scale1_kernel(xw_ref, dm_ref, o_ref):
    dinv = _dinv_from(dm_ref[0, 0], dm_ref[0, 1])
    o_ref[0] = xw_ref[...] * dinv


def _scale1(xW, dmparts):
    return pl.pallas_call(
        _scale1_kernel,
        grid=(3, NB),
        in_specs=[
            pl.BlockSpec((BN, D), lambda t, n: (n, 0)),
            pl.BlockSpec((1, NC, BN), lambda t, n: (t, 0, n)),
        ],
        out_specs=pl.BlockSpec((1, BN, D), lambda t, n: (t, n, 0)),
        out_shape=jax.ShapeDtypeStruct((3, NPAD, D), jnp.float32),
    )(xW, dmparts)


def _stepA_kernel(acc_ref, xw_ref, dm_ref, w2_ref, b1_ref, o_ref):
    dinv = _dinv_from(dm_ref[0, 0], dm_ref[0, 1])
    a = acc_ref[0] + acc_ref[1]
    h1 = jnp.maximum(a * dinv + xw_ref[...] * (dinv * dinv) + b1_ref[...], 0.0)
    o_ref[...] = jnp.dot(h1, w2_ref[...], preferred_element_type=jnp.float32) * dinv


def _stepA(t, acc1, xW, dmparts, W2, b1r):
    return pl.pallas_call(
        _stepA_kernel,
        grid=(NB,),
        in_specs=[
            pl.BlockSpec((NC, BN, D), lambda n: (0, n, 0)),
            pl.BlockSpec((BN, D), lambda n: (n, 0)),
            pl.BlockSpec((1, NC, BN), lambda n, _t=t: (_t, 0, n)),
            pl.BlockSpec((D, D), lambda n: (0, 0)),
            pl.BlockSpec((1, D), lambda n: (0, 0)),
        ],
        out_specs=pl.BlockSpec((BN, D), lambda n: (n, 0)),
        out_shape=jax.ShapeDtypeStruct((NPAD, D), jnp.float32),
    )(acc1, xW, dmparts, W2, b1r)


def _stepB_kernel(acc_ref, t2s_ref, dm_ref, mk_ref, b2_ref, o_ref):
    dinv = _dinv_from(dm_ref[0, 0], dm_ref[0, 1])
    a = acc_ref[0] + acc_ref[1]
    h = (a + t2s_ref[...]) * dinv + b2_ref[...]
    mcnt = (mk_ref[0, 0] + mk_ref[0, 1]).reshape(-1, 1)
    o_ref[...] = jnp.where(mcnt > 0.0, 0.0, h)


def _stepB(t, acc2, t2s, dmparts, b2r):
    return pl.pallas_call(
        _stepB_kernel,
        grid=(NB,),
        in_specs=[
            pl.BlockSpec((NC, BN, D), lambda n: (0, n, 0)),
            pl.BlockSpec((BN, D), lambda n: (n, 0)),
            pl.BlockSpec((1, NC, BN), lambda n, _t=t: (_t, 0, n)),
            pl.BlockSpec((1, NC, BN), lambda n, _t=t: (3 + _t, 0, n)),
            pl.BlockSpec((1, D), lambda n: (0, 0)),
        ],
        out_specs=pl.BlockSpec((BN, D), lambda n: (n, 0)),
        out_shape=jax.ShapeDtypeStruct((NPAD, D), jnp.float32),
    )(acc2, t2s, dmparts, dmparts, b2r)


def _mixer_kernel(bn, h0_ref, h1_ref, h2_ref, wt1_ref, bt1_ref, wt2_ref,
                  bt2_ref, tng_ref, tnb_ref, cng_ref, cnb_ref,
                  wc1_ref, bc1_ref, wc2_ref, bc2_ref, o_ref):
    hl = [h0_ref[...], h1_ref[...], h2_ref[...]]
    tng, tnb = tng_ref[...], tnb_ref[...]
    cng, cnb = cng_ref[...], cnb_ref[...]
    wc1, wc2 = wc1_ref[...], wc2_ref[...]
    bc1, bc2 = bc1_ref[...], bc2_ref[...]
    lnk = [_ln(h, tng, tnb) for h in hl]
    for i in (1, 2, 3):
        u = []
        for j in range(M):
            cij = wt1_ref[0, j]
            for m in range(1, M - i):
                cij = cij + wt1_ref[m, j]
            pre = jnp.broadcast_to(tnb * cij + bt1_ref[j], (bn, D))
            for k in range(i):
                pre = pre + lnk[k] * wt1_ref[M - i + k, j]
            u.append(_gelu(pre))
        acc = jnp.zeros((bn, D), jnp.float32)
        for m in range(M):
            ym = u[0] * wt2_ref[0, m]
            for j in range(1, M):
                ym = ym + u[j] * wt2_ref[j, m]
            ym = ym + bt2_ref[m]
            x1 = ym + hl[m - (M - i)] if m >= M - i else ym
            z = jnp.dot(_gelu(jnp.dot(_ln(x1, cng, cnb), wc1,
                                      preferred_element_type=jnp.float32) + bc1),
                        wc2, preferred_element_type=jnp.float32) + bc2
            acc = acc + x1 + z
        o_ref[i - 1] = acc * np.float32(1.0 / M)


def _mixer(h0, h1, h2, Wt1, bt1, Wt2, bt2, tng, tnb, cng, cnb, Wc1, bc1r, Wc2, bc2r):
    n = h0.shape[0]
    bn = min(BN, n)
    grid = n // bn
    smem = pl.BlockSpec(memory_space=pltpu.SMEM)
    full = lambda shp: pl.BlockSpec(shp, lambda i: tuple(0 for _ in shp))
    return pl.pallas_call(
        functools.partial(_mixer_kernel, bn),
        grid=(grid,),
        in_specs=[
            pl.BlockSpec((bn, D), lambda i: (i, 0)),
            pl.BlockSpec((bn, D), lambda i: (i, 0)),
            pl.BlockSpec((bn, D), lambda i: (i, 0)),
            smem, smem, smem, smem,
            full((1, D)), full((1, D)), full((1, D)), full((1, D)),
            full((D, D)), full((1, D)), full((D, D)), full((1, D)),
        ],
        out_specs=pl.BlockSpec((3, bn, D), lambda i: (0, i, 0)),
        out_shape=jax.ShapeDtypeStruct((3, n, D), jnp.float32),
    )(h0, h1, h2, Wt1, bt1, Wt2, bt2, tng, tnb, cng, cnb, Wc1, bc1r, Wc2, bc2r)


def kernel(emb, W1, b1, W2, b2, tn_g, tn_b, Wt1, bt1, Wt2, bt2,
           cn_g, cn_b, Wc1, bc1, Wc2, bc2, node_t, edge_index_t):
    f32 = jnp.float32
    ei = edge_index_t.astype(jnp.int32)
    nt = node_t.astype(jnp.int32)

    b1r = b1.reshape(1, D)
    b2r = b2.reshape(1, D)
    bt1r = bt1
    tngr, tnbr = tn_g.reshape(1, D), tn_b.reshape(1, D)
    cngr, cnbr = cn_g.reshape(1, D), cn_b.reshape(1, D)
    bc1r, bc2r = bc1.reshape(1, D), bc2.reshape(1, D)

    dm_idx = jnp.concatenate([
        ei[0, 1], ei[1, 1] + NP, ei[2, 1] + 2 * NP,
        nt[0] + 3 * NP, nt[1] + 4 * NP, nt[2] + 5 * NP,
        jnp.full((DM_PAD,), JUNK, jnp.int32),
    ]).reshape(NW, DM_NCHUNK, CH)
    dm_out = _build_sc_degmask()(dm_idx,
                         jnp.ones((CH,), f32),
                         jnp.zeros((DM_ROWS,), f32))
    dmparts = dm_out.reshape(NC, SEG, NP).swapaxes(0, 1)

    embp = jnp.concatenate([emb, jnp.zeros((NPAD - N, D), f32)], axis=0)
    xW = _matmul(embp, W1)
    tab1 = _scale1(xW, dmparts)

    z_agg = jnp.zeros((CH, D), f32)
    hs = []
    for t in range(3):
        src = ei[t, 0].reshape(NW, NCHUNK, CH)
        dst = ei[t, 1].reshape(NW, NCHUNK, CH)
        acc1 = _build_sc_aggregate()(tab1[t], src, dst, z_agg).reshape(NC, NPAD, D)
        t2s = _stepA(t, acc1, xW, dmparts, W2, b1r)
        acc2 = _build_sc_aggregate()(t2s, src, dst, z_agg).reshape(NC, NPAD, D)
        hs.append(_stepB(t, acc2, t2s, dmparts, b2r))

    outs = _mixer(hs[0], hs[1], hs[2], Wt1, bt1r, Wt2, bt2,
                  tngr, tnbr, cngr, cnbr, Wc1, bc1r, Wc2, bc2r)
    zrow = jnp.zeros((8, D), f32)
    out0 = _mixer(zrow, zrow, zrow, Wt1, bt1r, Wt2, bt2,
                  tngr, tnbr, cngr, cnbr, Wc1, bc1r, Wc2, bc2r)[2, 0]

    return jnp.concatenate(
        [jnp.broadcast_to(out0, (1, N, D)), outs[:, :N]], axis=0)
